# baseline (speedup 1.0000x reference)
import functools

import jax
import jax.numpy as jnp
from jax import lax
from jax.experimental import pallas as pl
from jax.experimental.pallas import tpu as pltpu
from jax.experimental.pallas import tpu_sc as plsc

SEQ = 200
DIM = 64
NC = 2
NS = 16
NW = NC * NS

C = 128
NG = DIM // 16
NB = 2
K = 1


def _emb_body(idx_hbm, tok2_hbm, pos2_hbm, out_hbm, idx_v, idx2_v, sup_v,
              out_v, pos2_v, gsem, osem):
    total = idx_hbm.shape[0]
    per_w = total // NW
    nchunks = per_w // C

    wid = lax.axis_index("s") * NC + lax.axis_index("c")
    base_w = wid * per_w

    pltpu.sync_copy(pos2_hbm, pos2_v)

    def start_gather(g, b):
        base = base_w + g * C
        pltpu.sync_copy(idx_hbm.at[pl.ds(base, C)], idx_v.at[b])
        for i in range(C // 16):
            idx2_v[b, pl.ds(i * 16, 16)] = (
                lax.shift_right_logical(idx_v[b, pl.ds(i * 16, 16)], 1)
            )
        pltpu.async_copy(tok2_hbm.at[idx2_v.at[b]], sup_v.at[b], gsem.at[b])

    def wait_gather(b):
        pltpu.make_async_copy(
            tok2_hbm.at[idx2_v.at[b]], sup_v.at[b], gsem.at[b]
        ).wait()

    def start_out(g, b):
        base = base_w + g * C
        pltpu.async_copy(
            out_v.at[b], out_hbm.at[pl.ds(base, C), :], osem.at[b]
        )

    def wait_out(b):
        pltpu.make_async_copy(
            out_v.at[b], out_hbm.at[pl.ds(0, C), :], osem.at[b]
        ).wait()

    for b in range(K):
        start_gather(b, b)

    def outer(h, carry):
        for b in range(NB):
            g = h * NB + b
            wait_gather(b)
            s0 = lax.rem(g * C, SEQ)

            def add_body(q, c2):
                j0 = q * 16
                cv = (idx_v[b, pl.ds(j0, 16)] & 1) * DIM
                for l in range(16):
                    j = j0 + l
                    col0 = cv[l]
                    for d in range(NG):
                        out_v[b, j, pl.ds(d * 16, 16)] = (
                            sup_v[b, j, pl.ds(col0 + d * 16, 16)]
                            + pos2_v[s0 + j, pl.ds(d * 16, 16)]
                        )
                return c2

            lax.fori_loop(0, C // 16, add_body, 0)
            start_out(g, b)

            b2 = (b + K) % NB

            @pl.when(g + K < nchunks)
            def _():
                start_gather(g + K, b2)

            @pl.when(g + K >= NB)
            def _():
                wait_out(b2)
        return carry

    lax.fori_loop(0, nchunks // NB, outer, 0)

    for g in range(nchunks - K, nchunks):
        wait_out(g % NB)


@functools.partial(jax.jit, static_argnames=())
def _emb(idx_flat, tok2, pos2):
    total = idx_flat.shape[0]
    run = pl.kernel(
        _emb_body,
        out_type=jax.ShapeDtypeStruct((total, DIM), jnp.float32),
        mesh=plsc.VectorSubcoreMesh(core_axis_name="c", subcore_axis_name="s"),
        scratch_types=[
            pltpu.VMEM((NB, C), jnp.int32),
            pltpu.VMEM((NB, C), jnp.int32),
            pltpu.VMEM((NB, C, 2 * DIM), jnp.float32),
            pltpu.VMEM((NB, C, DIM), jnp.float32),
            pltpu.VMEM((2 * SEQ, DIM), jnp.float32),
            pltpu.SemaphoreType.DMA((NB,)),
            pltpu.SemaphoreType.DMA((NB,)),
        ],
        compiler_params=pltpu.CompilerParams(use_tc_tiling_on_sc=True),
    )
    return run(idx_flat, tok2, pos2)


def kernel(inputs, token_table, pos_table):
    batch, seq = inputs.shape
    flat = inputs.reshape(-1).astype(jnp.int32)
    tok2 = token_table.reshape(-1, 2 * DIM)
    pos2 = jnp.concatenate([pos_table, pos_table], axis=0)
    out = _emb(flat, tok2, pos2)
    return out.reshape(batch, seq, DIM)


# speedup vs baseline: 1.3433x; 1.3433x over previous
import functools

import jax
import jax.numpy as jnp
from jax import lax
from jax.experimental import pallas as pl
from jax.experimental.pallas import tpu as pltpu
from jax.experimental.pallas import tpu_sc as plsc

SEQ = 200
DIM = 64
NC = 2
NS = 16
NW = NC * NS

C = 80
NG = DIM // 16
NB = 4
K = 2


def _emb_body(idx_hbm, tokp_hbm, pos_hbm, out_hbm, idx_v, sup_v, out_v,
              pos_v, gsem, osem):
    total = idx_hbm.shape[0]
    per_w = total // NW
    nchunks = per_w // C

    wid = lax.axis_index("s") * NC + lax.axis_index("c")
    base_w = wid * per_w

    pltpu.sync_copy(pos_hbm, pos_v)

    def start_gather(g, b):
        base = base_w + g * C
        pltpu.sync_copy(idx_hbm.at[pl.ds(base, C)], idx_v.at[b])
        pltpu.async_copy(tokp_hbm.at[idx_v.at[b]], sup_v.at[b], gsem.at[b])

    def wait_gather(b):
        pltpu.make_async_copy(
            tokp_hbm.at[idx_v.at[b]], sup_v.at[b], gsem.at[b]
        ).wait()

    def start_out(g, b):
        base = base_w + g * C
        pltpu.async_copy(
            out_v.at[b], out_hbm.at[pl.ds(base, C), :], osem.at[b]
        )

    def wait_out(b):
        pltpu.make_async_copy(
            out_v.at[b], out_hbm.at[pl.ds(0, C), :], osem.at[b]
        ).wait()

    for b in range(K):
        start_gather(b, b)

    def outer(h, carry):
        for b in range(NB):
            g = h * NB + b
            wait_gather(b)
            s0 = lax.rem(g * C, SEQ)

            def add_body(j, c2):
                s = s0 + j
                s = jnp.where(s >= SEQ, s - SEQ, s)
                for d in range(NG):
                    out_v[b, j, pl.ds(d * 16, 16)] = (
                        sup_v[b, j, pl.ds(d * 16, 16)]
                        + pos_v[s, pl.ds(d * 16, 16)]
                    )
                return c2

            lax.fori_loop(0, C, add_body, 0)
            start_out(g, b)

            b2 = (b + K) % NB

            @pl.when(g + K >= NB)
            def _():
                wait_out(b2)

            @pl.when(g + K < nchunks)
            def _():
                start_gather(g + K, b2)
        return carry

    lax.fori_loop(0, nchunks // NB, outer, 0)

    for g in range(nchunks - K, nchunks):
        wait_out(g % NB)


@functools.partial(jax.jit, static_argnames=())
def _emb(idx_flat, tokp, pos):
    total = idx_flat.shape[0]
    run = pl.kernel(
        _emb_body,
        out_type=jax.ShapeDtypeStruct((total, DIM), jnp.float32),
        mesh=plsc.VectorSubcoreMesh(core_axis_name="c", subcore_axis_name="s"),
        scratch_types=[
            pltpu.VMEM((NB, C), jnp.int32),
            pltpu.VMEM((NB, C, 2 * DIM), jnp.float32),
            pltpu.VMEM((NB, C, DIM), jnp.float32),
            pltpu.VMEM((SEQ, DIM), jnp.float32),
            pltpu.SemaphoreType.DMA((NB,)),
            pltpu.SemaphoreType.DMA((NB,)),
        ],
        compiler_params=pltpu.CompilerParams(use_tc_tiling_on_sc=True),
    )
    return run(idx_flat, tokp, pos)


def kernel(inputs, token_table, pos_table):
    batch, seq = inputs.shape
    flat = inputs.reshape(-1).astype(jnp.int32)
    tokp = jnp.pad(token_table, ((0, 0), (0, DIM)))
    out = _emb(flat, tokp, pos_table)
    return out.reshape(batch, seq, DIM)


# speedup vs baseline: 1.5378x; 1.1447x over previous
import functools

import jax
import jax.numpy as jnp
from jax import lax
from jax.experimental import pallas as pl
from jax.experimental.pallas import tpu as pltpu
from jax.experimental.pallas import tpu_sc as plsc

SEQ = 200
DIM = 64
NC = 2
NS = 16
NW = NC * NS

C = 64
NG = DIM // 16
NB = 4
K = 2

PAD_ROWS = 4000


def _pad_body(t_ref, o_ref):
    o_ref[:, :DIM] = t_ref[...]
    o_ref[:, DIM:] = t_ref[...]


def _widen_table(token_table):
    vocab = token_table.shape[0]
    return pl.pallas_call(
        _pad_body,
        grid=(vocab // PAD_ROWS,),
        in_specs=[pl.BlockSpec((PAD_ROWS, DIM), lambda i: (i, 0))],
        out_specs=pl.BlockSpec((PAD_ROWS, 2 * DIM), lambda i: (i, 0)),
        out_shape=jax.ShapeDtypeStruct((vocab, 2 * DIM), jnp.float32),
    )(token_table)


def _emb_body(idx_hbm, tokp_hbm, pos_hbm, out_hbm, idx_v, sup_v, out_v,
              pos_v, gsem, osem):
    total = idx_hbm.shape[0]
    per_w = total // NW
    nchunks = per_w // C

    wid = lax.axis_index("s") * NC + lax.axis_index("c")
    base_w = wid * per_w

    pltpu.sync_copy(pos_hbm, pos_v)
    pltpu.sync_copy(idx_hbm.at[pl.ds(base_w, per_w)], idx_v)

    def start_gather(g, b):
        pltpu.async_copy(
            tokp_hbm.at[idx_v.at[pl.ds(g * C, C)]], sup_v.at[b], gsem.at[b]
        )

    def wait_gather(g, b):
        pltpu.make_async_copy(
            tokp_hbm.at[idx_v.at[pl.ds(g * C, C)]], sup_v.at[b], gsem.at[b]
        ).wait()

    def start_out(g, b):
        base = base_w + g * C
        pltpu.async_copy(
            out_v.at[b], out_hbm.at[pl.ds(base, C), :], osem.at[b]
        )

    def wait_out(b):
        pltpu.make_async_copy(
            out_v.at[b], out_hbm.at[pl.ds(0, C), :], osem.at[b]
        ).wait()

    for b in range(K):
        start_gather(b, b)

    def outer(h, carry):
        for b in range(NB):
            g = h * NB + b
            wait_gather(g, b)
            s0 = lax.rem(g * C, SEQ)

            def add_body(j, c2):
                s = s0 + j
                s = jnp.where(s >= SEQ, s - SEQ, s)
                for d in range(NG):
                    out_v[b, j, pl.ds(d * 16, 16)] = (
                        sup_v[b, j, pl.ds(d * 16, 16)]
                        + pos_v[s, pl.ds(d * 16, 16)]
                    )
                return c2

            lax.fori_loop(0, C, add_body, 0)
            start_out(g, b)

            b2 = (b + K) % NB

            @pl.when(g + K >= NB)
            def _():
                wait_out(b2)

            @pl.when(g + K < nchunks)
            def _():
                start_gather(g + K, b2)
        return carry

    lax.fori_loop(0, nchunks // NB, outer, 0)

    for g in range(nchunks - K, nchunks):
        wait_out(g % NB)


@functools.partial(jax.jit, static_argnames=())
def _emb(idx_flat, tokp, pos):
    total = idx_flat.shape[0]
    per_w = total // NW
    run = pl.kernel(
        _emb_body,
        out_type=jax.ShapeDtypeStruct((total, DIM), jnp.float32),
        mesh=plsc.VectorSubcoreMesh(core_axis_name="c", subcore_axis_name="s"),
        scratch_types=[
            pltpu.VMEM((per_w,), jnp.int32),
            pltpu.VMEM((NB, C, 2 * DIM), jnp.float32),
            pltpu.VMEM((NB, C, DIM), jnp.float32),
            pltpu.VMEM((SEQ, DIM), jnp.float32),
            pltpu.SemaphoreType.DMA((NB,)),
            pltpu.SemaphoreType.DMA((NB,)),
        ],
        compiler_params=pltpu.CompilerParams(use_tc_tiling_on_sc=True),
    )
    return run(idx_flat, tokp, pos)


def kernel(inputs, token_table, pos_table):
    batch, seq = inputs.shape
    flat = inputs.reshape(-1).astype(jnp.int32)
    tokp = _widen_table(token_table)
    out = _emb(flat, tokp, pos_table)
    return out.reshape(batch, seq, DIM)
